# baseline (speedup 1.0000x reference)
import functools

import jax
import jax.numpy as jnp
from jax import lax
from jax.experimental import pallas as pl
from jax.experimental.pallas import tpu as pltpu
from jax.experimental.pallas import tpu_sc as plsc
import numpy as np

B, V, S, F, K = 4, 1024, 16, 128, 16
MAXD = float(np.finfo(np.float32).max)
R = 256


def _topk_body(act_ref, rows_ref, cols_ref, vals_ref, idx_ref):
    b = pl.program_id(0)
    rblk = pl.program_id(1)
    act = act_ref[0, 0, 0]
    cr = rows_ref[0]
    cc = cols_ref[0]

    def _tree8(h):
        return ((h[0] + h[4]) + (h[2] + h[6])) + ((h[1] + h[5]) + (h[3] + h[7]))

    NT = V // 128
    lane = lax.broadcasted_iota(jnp.int32, (R, 128), 1)
    row = rblk * R + lax.broadcasted_iota(jnp.int32, (R, 128), 0)
    inf = jnp.float32(jnp.inf)
    row_ok = row < act

    dist_t = []
    for t in range(NT):
        a = [
            jnp.abs(cr[:, s : s + 1] - cc[s : s + 1, t * 128 : (t + 1) * 128])
            for s in range(S)
        ]
        d = _tree8(a[0:8]) + _tree8(a[8:16])
        colj = t * 128 + lane
        d = jnp.where(row_ok & (colj < act), d, MAXD)
        d = jnp.where(row == colj, inf, d)
        dist_t.append(d)

    vals_cols = []
    idx_cols = []
    mi = None
    for _ in range(K - 1):
        mvals = []
        for t in range(NT):
            d = dist_t[t]
            if mi is not None:
                d = jnp.where(t * 128 + lane == mi, inf, d)
                dist_t[t] = d
            mvals.append(d)
        mts = list(range(NT))
        while len(mvals) > 1:
            nxt_v, nxt_t = [], []
            for i in range(0, len(mvals), 2):
                va, vb = mvals[i], mvals[i + 1]
                lt = vb < va
                nxt_v.append(jnp.minimum(va, vb))
                nxt_t.append(jnp.where(lt, mts[i + 1], mts[i]))
            mvals, mts = nxt_v, nxt_t
        mkey, mt = mvals[0], mts[0]
        mv = jnp.min(mkey, axis=1, keepdims=True)
        mi = jnp.min(
            jnp.where(mkey == mv, mt * 128 + lane, V), axis=1, keepdims=True
        )
        vals_cols.append(mv)
        idx_cols.append(mi)
    vals15 = jnp.concatenate(vals_cols, axis=1)
    idx15 = jnp.concatenate(idx_cols, axis=1)
    row15 = rblk * R + lax.broadcasted_iota(jnp.int32, (R, K - 1), 0)
    j15 = lax.broadcasted_iota(jnp.int32, (R, K - 1), 1)
    inactive = row15 >= act
    vals_ref[0] = jnp.where(inactive, MAXD, vals15)
    idx_ref[0] = jnp.where(inactive, j15 + 1, idx15)


def _topk_call(coords, coords_t, active):
    nb = coords.shape[0]
    return pl.pallas_call(
        _topk_body,
        grid=(nb, V // R),
        in_specs=[
            pl.BlockSpec((1, 1, 1), lambda b, r: (b, 0, 0), memory_space=pltpu.SMEM),
            pl.BlockSpec((1, R, S), lambda b, r: (b, r, 0)),
            pl.BlockSpec((1, S, V), lambda b, r: (b, 0, 0)),
        ],
        out_specs=[
            pl.BlockSpec((1, R, K - 1), lambda b, r: (b, r, 0)),
            pl.BlockSpec((1, R, K - 1), lambda b, r: (b, r, 0)),
        ],
        out_shape=[
            jax.ShapeDtypeStruct((nb, V, K - 1), jnp.float32),
            jax.ShapeDtypeStruct((nb, V, K - 1), jnp.int32),
        ],
        compiler_params=pltpu.CompilerParams(
            dimension_semantics=("parallel", "parallel"),
        ),
    )(active.reshape(nb, 1, 1), coords, coords_t)


NIDXB = V * (K - 1)
NC, NS = 2, 16
NW = NC * NS
BPW = NIDXB // NW


def _gather_call(table, idx_flat):
    mesh = plsc.VectorSubcoreMesh(core_axis_name="c", subcore_axis_name="s")

    @functools.partial(
        pl.kernel,
        mesh=mesh,
        out_type=jax.ShapeDtypeStruct((NIDXB, F), jnp.float32),
        scratch_types=[
            pltpu.VMEM((BPW,), jnp.int32),
            pltpu.VMEM((BPW, F), jnp.float32),
            pltpu.SemaphoreType.DMA,
        ],
    )
    def gk(table_hbm, idx_hbm, out_hbm, idx_v, buf, sem):
        wid = lax.axis_index("s") * NC + lax.axis_index("c")
        base = wid * BPW
        pltpu.sync_copy(idx_hbm.at[pl.ds(base, BPW)], idx_v)
        pltpu.async_copy(table_hbm.at[idx_v], buf, sem).wait()
        pltpu.sync_copy(buf, out_hbm.at[pl.ds(base, BPW)])

    return gk(table, idx_flat)


def kernel(coordinates, features, active_vertices):
    coords_t = jnp.transpose(coordinates, (0, 2, 1))
    nds = []
    nfs = []
    for b in range(B):
        nd_b, idx_b = _topk_call(
            coordinates[b : b + 1],
            coords_t[b : b + 1],
            active_vertices[b : b + 1],
        )
        nf_b = _gather_call(features[b], idx_b.reshape(NIDXB))
        nds.append(nd_b)
        nfs.append(nf_b.reshape(1, V, K - 1, F))
    neighbour_distances = jnp.concatenate(nds, axis=0)
    neighbour_features = jnp.concatenate(nfs, axis=0)
    return (neighbour_distances, neighbour_features)
